# baseline (device time: 384207 ns/iter reference)
import jax
import jax.numpy as jnp
from jax import lax
from jax.experimental import pallas as pl
from jax.experimental.pallas import tpu as pltpu

N_DEV = 8
SCALE = 128 ** -0.5


def kernel(Q, K, V):
    b, s, h, d = Q.shape
    hh = h // 2
    qh = jnp.transpose((Q[0] * SCALE).astype(jnp.bfloat16), (1, 0, 2))
    kh = jnp.transpose(K[0].astype(jnp.bfloat16), (1, 0, 2))
    vh = jnp.transpose(V[0].astype(jnp.bfloat16), (1, 0, 2))

    def body(q_ref, k_ref, v_ref, out_ref,
             cw_ref, ccw_ref, cw_send, cw_recv, ccw_send, ccw_recv, l_ref):
        my = lax.axis_index("i")
        left = (my - 1) % N_DEV
        right = (my + 1) % N_DEV

        barrier = pltpu.get_barrier_semaphore()
        for nbr in (left, right):
            pl.semaphore_signal(
                barrier, inc=1, device_id=(nbr,),
                device_id_type=pl.DeviceIdType.MESH,
            )
        pl.semaphore_wait(barrier, 2)

        cw_ref[0, 0] = k_ref[:hh]
        cw_ref[0, 1] = v_ref[:hh]
        ccw_ref[0, 0] = k_ref[hh:]
        ccw_ref[0, 1] = v_ref[hh:]
        l_ref[...] = jnp.zeros(l_ref.shape, l_ref.dtype)
        out_ref[...] = jnp.zeros(out_ref.shape, out_ref.dtype)

        def consume(buf_ref, slot, head_off):
            def per_head(i, carry):
                hd = i + head_off
                q = q_ref[hd]
                k = buf_ref[slot, 0, i]
                v = buf_ref[slot, 1, i]
                s_mat = lax.dot_general(
                    q, k, (((1,), (1,)), ((), ())),
                    preferred_element_type=jnp.float32,
                )
                p = jnp.exp(s_mat)
                l_ref[hd] += jnp.sum(p, axis=1, keepdims=True)
                out_ref[hd] += lax.dot_general(
                    p.astype(jnp.bfloat16), v, (((1,), (0,)), ((), ())),
                    preferred_element_type=jnp.float32,
                )
                return carry

            lax.fori_loop(0, hh, per_head, 0)

        hq = hh // 2
        for hop in range(N_DEV - 1):
            snd = hop % 2
            rcv = (hop + 1) % 2
            rdmas = []
            for sub, (h0, h1) in enumerate(((0, hq), (hq, hh))):
                rdmas.append(pltpu.make_async_remote_copy(
                    src_ref=cw_ref.at[snd, :, h0:h1],
                    dst_ref=cw_ref.at[rcv, :, h0:h1],
                    send_sem=cw_send.at[snd, sub],
                    recv_sem=cw_recv.at[rcv, sub],
                    device_id=(right,),
                    device_id_type=pl.DeviceIdType.MESH,
                ))
                rdmas.append(pltpu.make_async_remote_copy(
                    src_ref=ccw_ref.at[snd, :, h0:h1],
                    dst_ref=ccw_ref.at[rcv, :, h0:h1],
                    send_sem=ccw_send.at[snd, sub],
                    recv_sem=ccw_recv.at[rcv, sub],
                    device_id=(left,),
                    device_id_type=pl.DeviceIdType.MESH,
                ))
            for rd in rdmas:
                rd.start()
            consume(cw_ref, snd, 0)
            consume(ccw_ref, snd, hh)
            for rd in rdmas:
                rd.wait()
        last = (N_DEV - 1) % 2
        consume(cw_ref, last, 0)
        consume(ccw_ref, last, hh)

        def norm(hd, carry):
            out_ref[hd] = out_ref[hd] / l_ref[hd]
            return carry

        lax.fori_loop(0, h, norm, 0)

    out = pl.pallas_call(
        body,
        out_shape=jax.ShapeDtypeStruct((h, s, d), jnp.float32),
        in_specs=[pl.BlockSpec(memory_space=pltpu.VMEM)] * 3,
        out_specs=pl.BlockSpec(memory_space=pltpu.VMEM),
        scratch_shapes=[
            pltpu.VMEM((2, 2, hh, s, d), jnp.bfloat16),
            pltpu.VMEM((2, 2, hh, s, d), jnp.bfloat16),
            pltpu.SemaphoreType.DMA((2, 2)),
            pltpu.SemaphoreType.DMA((2, 2)),
            pltpu.SemaphoreType.DMA((2, 2)),
            pltpu.SemaphoreType.DMA((2, 2)),
            pltpu.VMEM((h, s, 1), jnp.float32),
        ],
        compiler_params=pltpu.CompilerParams(collective_id=0),
    )(qh, kh, vh)
    return jnp.transpose(out, (1, 0, 2))[None]


# device time: 376147 ns/iter; 1.0214x vs baseline; 1.0214x over previous
import jax
import jax.numpy as jnp
from jax import lax
from jax.experimental import pallas as pl
from jax.experimental.pallas import tpu as pltpu

N_DEV = 8
SCALE = 128 ** -0.5


def kernel(Q, K, V):
    b, s, h, d = Q.shape
    hh = h // 2
    qh = jnp.transpose((Q[0] * SCALE).astype(jnp.bfloat16), (1, 0, 2))
    kh = jnp.transpose(K[0].astype(jnp.bfloat16), (1, 0, 2))
    vh = jnp.transpose(V[0].astype(jnp.bfloat16), (1, 0, 2))

    def body(q_ref, k_ref, v_ref, out_ref,
             cw_ref, ccw_ref, cw_send, cw_recv, ccw_send, ccw_recv, l_ref):
        my = lax.axis_index("i")
        left = (my - 1) % N_DEV
        right = (my + 1) % N_DEV

        barrier = pltpu.get_barrier_semaphore()
        for nbr in (left, right):
            pl.semaphore_signal(
                barrier, inc=1, device_id=(nbr,),
                device_id_type=pl.DeviceIdType.MESH,
            )
        pl.semaphore_wait(barrier, 2)

        cw_ref[0, 0] = k_ref[:hh]
        cw_ref[0, 1] = v_ref[:hh]
        ccw_ref[0, 0] = k_ref[hh:]
        ccw_ref[0, 1] = v_ref[hh:]
        l_ref[...] = jnp.zeros(l_ref.shape, l_ref.dtype)
        out_ref[...] = jnp.zeros(out_ref.shape, out_ref.dtype)

        def consume(buf_ref, slot, head_off, i0=0, i1=None, final=False):
            def per_head(i, carry):
                hd = i + head_off
                q = q_ref[hd]
                k = buf_ref[slot, 0, i]
                v = buf_ref[slot, 1, i]
                s_mat = lax.dot_general(
                    q, k, (((1,), (1,)), ((), ())),
                    preferred_element_type=jnp.float32,
                )
                p = jnp.exp(s_mat)
                pv = lax.dot_general(
                    p.astype(jnp.bfloat16), v, (((1,), (0,)), ((), ())),
                    preferred_element_type=jnp.float32,
                )
                l_new = l_ref[hd] + jnp.sum(p, axis=1, keepdims=True)
                if final:
                    out_ref[hd] = (out_ref[hd] + pv) / l_new
                else:
                    l_ref[hd] = l_new
                    out_ref[hd] += pv
                return carry

            lax.fori_loop(i0, hh if i1 is None else i1, per_head, 0)

        hq = hh // 2
        for hop in range(N_DEV - 1):
            snd = hop % 2
            rcv = (hop + 1) % 2
            rdmas = []
            for sub, (h0, h1) in enumerate(((0, hq), (hq, hh))):
                rdmas.append(pltpu.make_async_remote_copy(
                    src_ref=cw_ref.at[snd, :, h0:h1],
                    dst_ref=cw_ref.at[rcv, :, h0:h1],
                    send_sem=cw_send.at[snd, sub],
                    recv_sem=cw_recv.at[rcv, sub],
                    device_id=(right,),
                    device_id_type=pl.DeviceIdType.MESH,
                ))
                rdmas.append(pltpu.make_async_remote_copy(
                    src_ref=ccw_ref.at[snd, :, h0:h1],
                    dst_ref=ccw_ref.at[rcv, :, h0:h1],
                    send_sem=ccw_send.at[snd, sub],
                    recv_sem=ccw_recv.at[rcv, sub],
                    device_id=(left,),
                    device_id_type=pl.DeviceIdType.MESH,
                ))
            for rd in rdmas:
                rd.start()
            consume(cw_ref, snd, 0)
            consume(ccw_ref, snd, hh)
            if hop < N_DEV - 2:
                for rd in rdmas:
                    rd.wait()
            else:
                rdmas[0].wait()
                rdmas[1].wait()
                consume(cw_ref, rcv, 0, 0, hq, final=True)
                consume(ccw_ref, rcv, hh, 0, hq, final=True)
                rdmas[2].wait()
                rdmas[3].wait()
                consume(cw_ref, rcv, 0, hq, hh, final=True)
                consume(ccw_ref, rcv, hh, hq, hh, final=True)

    out = pl.pallas_call(
        body,
        out_shape=jax.ShapeDtypeStruct((h, s, d), jnp.float32),
        in_specs=[pl.BlockSpec(memory_space=pltpu.VMEM)] * 3,
        out_specs=pl.BlockSpec(memory_space=pltpu.VMEM),
        scratch_shapes=[
            pltpu.VMEM((2, 2, hh, s, d), jnp.bfloat16),
            pltpu.VMEM((2, 2, hh, s, d), jnp.bfloat16),
            pltpu.SemaphoreType.DMA((2, 2)),
            pltpu.SemaphoreType.DMA((2, 2)),
            pltpu.SemaphoreType.DMA((2, 2)),
            pltpu.SemaphoreType.DMA((2, 2)),
            pltpu.VMEM((h, s, 1), jnp.float32),
        ],
        compiler_params=pltpu.CompilerParams(collective_id=0),
    )(qh, kh, vh)
    return jnp.transpose(out, (1, 0, 2))[None]


# device time: 373346 ns/iter; 1.0291x vs baseline; 1.0075x over previous
import jax
import jax.numpy as jnp
from jax import lax
from jax.experimental import pallas as pl
from jax.experimental.pallas import tpu as pltpu

N_DEV = 8
SCALE = 128 ** -0.5


def kernel(Q, K, V):
    b, s, h, d = Q.shape
    hh = h // 2
    qh = jnp.transpose((Q[0] * SCALE).astype(jnp.bfloat16), (1, 0, 2))
    kh = jnp.transpose(K[0].astype(jnp.bfloat16), (1, 0, 2))
    vh = jnp.transpose(V[0].astype(jnp.bfloat16), (1, 0, 2))

    def body(q_ref, k_ref, v_ref, out_ref,
             cw_ref, ccw_ref, cw_send, cw_recv, ccw_send, ccw_recv, l_ref):
        my = lax.axis_index("i")
        left = (my - 1) % N_DEV
        right = (my + 1) % N_DEV

        barrier = pltpu.get_barrier_semaphore()
        for nbr in (left, right):
            pl.semaphore_signal(
                barrier, inc=1, device_id=(nbr,),
                device_id_type=pl.DeviceIdType.MESH,
            )
        pl.semaphore_wait(barrier, 2)

        def consume(buf_ref, slot, head_off, i0=0, i1=None, final=False):
            def per_head(i, carry):
                hd = i + head_off
                q = q_ref[hd]
                k = buf_ref[slot, 0, i]
                v = buf_ref[slot, 1, i]
                s_mat = lax.dot_general(
                    q, k, (((1,), (1,)), ((), ())),
                    preferred_element_type=jnp.float32,
                )
                p = jnp.exp(s_mat)
                pv = lax.dot_general(
                    p.astype(jnp.bfloat16), v, (((1,), (0,)), ((), ())),
                    preferred_element_type=jnp.float32,
                )
                l_new = l_ref[hd] + jnp.sum(p, axis=1, keepdims=True)
                if final:
                    out_ref[hd] = (out_ref[hd] + pv) / l_new
                else:
                    l_ref[hd] = l_new
                    out_ref[hd] += pv
                return carry

            lax.fori_loop(i0, hh if i1 is None else i1, per_head, 0)

        def consume_local():
            def per_head(hd, carry):
                q = q_ref[hd]
                k = k_ref[hd]
                v = v_ref[hd]
                s_mat = lax.dot_general(
                    q, k, (((1,), (1,)), ((), ())),
                    preferred_element_type=jnp.float32,
                )
                p = jnp.exp(s_mat)
                l_ref[hd] = jnp.sum(p, axis=1, keepdims=True)
                out_ref[hd] = lax.dot_general(
                    p.astype(jnp.bfloat16), v, (((1,), (0,)), ((), ())),
                    preferred_element_type=jnp.float32,
                )
                return carry

            lax.fori_loop(0, h, per_head, 0)

        hq = hh // 2
        for hop in range(N_DEV - 1):
            snd = hop % 2
            rcv = (hop + 1) % 2
            rdmas = []
            if hop == 0:
                for sub, ref in enumerate((k_ref, v_ref)):
                    rdmas.append(pltpu.make_async_remote_copy(
                        src_ref=ref.at[:hh],
                        dst_ref=cw_ref.at[rcv, sub],
                        send_sem=cw_send.at[snd, sub],
                        recv_sem=cw_recv.at[rcv, sub],
                        device_id=(right,),
                        device_id_type=pl.DeviceIdType.MESH,
                    ))
                    rdmas.append(pltpu.make_async_remote_copy(
                        src_ref=ref.at[hh:],
                        dst_ref=ccw_ref.at[rcv, sub],
                        send_sem=ccw_send.at[snd, sub],
                        recv_sem=ccw_recv.at[rcv, sub],
                        device_id=(left,),
                        device_id_type=pl.DeviceIdType.MESH,
                    ))
            else:
                for sub, (h0, h1) in enumerate(((0, hq), (hq, hh))):
                    rdmas.append(pltpu.make_async_remote_copy(
                        src_ref=cw_ref.at[snd, :, h0:h1],
                        dst_ref=cw_ref.at[rcv, :, h0:h1],
                        send_sem=cw_send.at[snd, sub],
                        recv_sem=cw_recv.at[rcv, sub],
                        device_id=(right,),
                        device_id_type=pl.DeviceIdType.MESH,
                    ))
                    rdmas.append(pltpu.make_async_remote_copy(
                        src_ref=ccw_ref.at[snd, :, h0:h1],
                        dst_ref=ccw_ref.at[rcv, :, h0:h1],
                        send_sem=ccw_send.at[snd, sub],
                        recv_sem=ccw_recv.at[rcv, sub],
                        device_id=(left,),
                        device_id_type=pl.DeviceIdType.MESH,
                    ))
            for rd in rdmas:
                rd.start()
            if hop == 0:
                consume_local()
            else:
                consume(cw_ref, snd, 0)
                consume(ccw_ref, snd, hh)
            if hop < N_DEV - 2:
                for rd in rdmas:
                    rd.wait()
            else:
                rdmas[0].wait()
                rdmas[1].wait()
                consume(cw_ref, rcv, 0, 0, hq, final=True)
                consume(ccw_ref, rcv, hh, 0, hq, final=True)
                rdmas[2].wait()
                rdmas[3].wait()
                consume(cw_ref, rcv, 0, hq, hh, final=True)
                consume(ccw_ref, rcv, hh, hq, hh, final=True)

    out = pl.pallas_call(
        body,
        out_shape=jax.ShapeDtypeStruct((h, s, d), jnp.float32),
        in_specs=[pl.BlockSpec(memory_space=pltpu.VMEM)] * 3,
        out_specs=pl.BlockSpec(memory_space=pltpu.VMEM),
        scratch_shapes=[
            pltpu.VMEM((2, 2, hh, s, d), jnp.bfloat16),
            pltpu.VMEM((2, 2, hh, s, d), jnp.bfloat16),
            pltpu.SemaphoreType.DMA((2, 2)),
            pltpu.SemaphoreType.DMA((2, 2)),
            pltpu.SemaphoreType.DMA((2, 2)),
            pltpu.SemaphoreType.DMA((2, 2)),
            pltpu.VMEM((h, s, 1), jnp.float32),
        ],
        compiler_params=pltpu.CompilerParams(collective_id=0),
    )(qh, kh, vh)
    return jnp.transpose(out, (1, 0, 2))[None]
